# baseline (device time: 204398 ns/iter reference)
import functools

import jax
import jax.numpy as jnp
from jax import lax
from jax.experimental import pallas as pl
from jax.experimental.pallas import tpu as pltpu

N_DEV = 16
M = 2048
M_CHUNK = M // N_DEV


def kernel(A, B):
    m, k_loc = A.shape
    _, n = B.shape

    def body(a_ref, b_ref, out_ref, send_buf, recv_buf, send_sems, recv_sems):
        me = lax.axis_index("i")

        barrier_sem = pltpu.get_barrier_semaphore()
        for p in range(1, N_DEV):
            pl.semaphore_signal(
                barrier_sem, inc=1,
                device_id=((me + p) % N_DEV,),
                device_id_type=pl.DeviceIdType.MESH,
            )
        pl.semaphore_wait(barrier_sem, N_DEV - 1)

        own = jnp.dot(
            a_ref[pl.ds(me * M_CHUNK, M_CHUNK), :], b_ref[:, :],
            preferred_element_type=jnp.float32,
        )
        recv_buf[me] = own

        sends = []
        for kstep in range(1, N_DEV):
            d = (me + kstep) % N_DEV
            send_buf[kstep] = jnp.dot(
                a_ref[pl.ds(d * M_CHUNK, M_CHUNK), :], b_ref[:, :],
                preferred_element_type=jnp.float32,
            )
            rdma = pltpu.make_async_remote_copy(
                src_ref=send_buf.at[kstep],
                dst_ref=recv_buf.at[me],
                send_sem=send_sems.at[kstep],
                recv_sem=recv_sems.at[me],
                device_id=(d,),
                device_id_type=pl.DeviceIdType.MESH,
            )
            rdma.start()
            sends.append(rdma)

        for kstep in range(1, N_DEV):
            j = (me - kstep) % N_DEV
            recv = pltpu.make_async_remote_copy(
                src_ref=send_buf.at[kstep],
                dst_ref=recv_buf.at[j],
                send_sem=send_sems.at[kstep],
                recv_sem=recv_sems.at[j],
                device_id=(j,),
                device_id_type=pl.DeviceIdType.MESH,
            )
            recv.wait_recv()

        for rdma in sends:
            rdma.wait_send()

        out_ref[:, :] = jnp.sum(recv_buf[:, :, :], axis=0)

    return pl.pallas_call(
        body,
        out_shape=jax.ShapeDtypeStruct((M_CHUNK, n), jnp.float32),
        in_specs=[
            pl.BlockSpec(memory_space=pltpu.VMEM),
            pl.BlockSpec(memory_space=pltpu.VMEM),
        ],
        out_specs=pl.BlockSpec(memory_space=pltpu.VMEM),
        scratch_shapes=[
            pltpu.VMEM((N_DEV, M_CHUNK, n), jnp.float32),
            pltpu.VMEM((N_DEV, M_CHUNK, n), jnp.float32),
            pltpu.SemaphoreType.DMA((N_DEV,)),
            pltpu.SemaphoreType.DMA((N_DEV,)),
        ],
        compiler_params=pltpu.CompilerParams(collective_id=0),
    )(A, B)


# device time: 101500 ns/iter; 2.0138x vs baseline; 2.0138x over previous
import functools

import jax
import jax.numpy as jnp
from jax import lax
from jax.experimental import pallas as pl
from jax.experimental.pallas import tpu as pltpu

N_DEV = 16
M = 2048
M_CHUNK = M // N_DEV


def kernel(A, B):
    m, k_loc = A.shape
    _, n = B.shape

    def body(a_ref, b_ref, out_ref, send_buf, recv_buf, send_sems, recv_sems):
        me = lax.axis_index("i")

        barrier_sem = pltpu.get_barrier_semaphore()
        for p in range(1, N_DEV):
            pl.semaphore_signal(
                barrier_sem, inc=1,
                device_id=((me + p) % N_DEV,),
                device_id_type=pl.DeviceIdType.MESH,
            )
        pl.semaphore_wait(barrier_sem, N_DEV - 1)

        own = jnp.dot(
            a_ref[pl.ds(me * M_CHUNK, M_CHUNK), :], b_ref[:, :],
            preferred_element_type=jnp.float32,
        )
        recv_buf[me] = own.astype(jnp.bfloat16)

        sends = []
        for kstep in range(1, N_DEV):
            d = (me + kstep) % N_DEV
            send_buf[kstep] = jnp.dot(
                a_ref[pl.ds(d * M_CHUNK, M_CHUNK), :], b_ref[:, :],
                preferred_element_type=jnp.float32,
            ).astype(jnp.bfloat16)
            rdma = pltpu.make_async_remote_copy(
                src_ref=send_buf.at[kstep],
                dst_ref=recv_buf.at[me],
                send_sem=send_sems.at[kstep],
                recv_sem=recv_sems.at[me],
                device_id=(d,),
                device_id_type=pl.DeviceIdType.MESH,
            )
            rdma.start()
            sends.append(rdma)

        for kstep in range(1, N_DEV):
            j = (me - kstep) % N_DEV
            recv = pltpu.make_async_remote_copy(
                src_ref=send_buf.at[kstep],
                dst_ref=recv_buf.at[j],
                send_sem=send_sems.at[kstep],
                recv_sem=recv_sems.at[j],
                device_id=(j,),
                device_id_type=pl.DeviceIdType.MESH,
            )
            recv.wait_recv()

        for rdma in sends:
            rdma.wait_send()

        out_ref[:, :] = jnp.sum(
            recv_buf[:, :, :].astype(jnp.float32), axis=0
        )

    return pl.pallas_call(
        body,
        out_shape=jax.ShapeDtypeStruct((M_CHUNK, n), jnp.float32),
        in_specs=[
            pl.BlockSpec(memory_space=pltpu.VMEM),
            pl.BlockSpec(memory_space=pltpu.VMEM),
        ],
        out_specs=pl.BlockSpec(memory_space=pltpu.VMEM),
        scratch_shapes=[
            pltpu.VMEM((N_DEV, M_CHUNK, n), jnp.bfloat16),
            pltpu.VMEM((N_DEV, M_CHUNK, n), jnp.bfloat16),
            pltpu.SemaphoreType.DMA((N_DEV,)),
            pltpu.SemaphoreType.DMA((N_DEV,)),
        ],
        compiler_params=pltpu.CompilerParams(collective_id=0),
    )(A, B)


# device time: 62859 ns/iter; 3.2517x vs baseline; 1.6147x over previous
import jax
import jax.numpy as jnp
from jax import lax
from jax.experimental import pallas as pl
from jax.experimental.pallas import tpu as pltpu

N_DEV = 16
N_PLANE = 4
N_Z = 4
M = 2048
M_CHUNK = M // N_DEV


def kernel(A, B):
    m, k_loc = A.shape
    _, n = B.shape

    def body(
        a_ref, b_ref, out_ref,
        send1, recv1, send2, recv2,
        send_sems1, recv_sems1, send_sems2, recv_sems2,
    ):
        me = lax.axis_index("i")
        z = me // N_PLANE
        c = me % N_PLANE

        barrier_sem = pltpu.get_barrier_semaphore()
        partners = [z * N_PLANE + (c + kp) % N_PLANE for kp in range(1, N_PLANE)]
        partners += [((z + kz) % N_Z) * N_PLANE + c for kz in range(1, N_Z)]
        for p in partners:
            pl.semaphore_signal(
                barrier_sem, inc=1,
                device_id=(p,), device_id_type=pl.DeviceIdType.MESH,
            )
        pl.semaphore_wait(barrier_sem, len(partners))

        def block(owner):
            return jnp.dot(
                a_ref[pl.ds(owner * M_CHUNK, M_CHUNK), :], b_ref[:, :],
                preferred_element_type=jnp.float32,
            )

        sends = []
        for zi in range(N_Z):
            zz = (z + 1 + zi) % N_Z
            recv1[zz, c] = block(zz * N_PLANE + c).astype(jnp.bfloat16)
            for kp in range(N_PLANE - 1):
                cp = (c + 1 + kp) % N_PLANE
                send1[kp, zi] = block(zz * N_PLANE + cp).astype(jnp.bfloat16)
                rdma = pltpu.make_async_remote_copy(
                    src_ref=send1.at[kp, zi],
                    dst_ref=recv1.at[zz, c],
                    send_sem=send_sems1.at[kp, zi],
                    recv_sem=recv_sems1.at[zz, c],
                    device_id=(z * N_PLANE + cp,),
                    device_id_type=pl.DeviceIdType.MESH,
                )
                rdma.start()
                sends.append(rdma)

        for zi in range(N_Z):
            zz = (z + 1 + zi) % N_Z
            for kp in range(N_PLANE - 1):
                cp = (c + 1 + kp) % N_PLANE
                recv = pltpu.make_async_remote_copy(
                    src_ref=send1.at[kp, zi],
                    dst_ref=recv1.at[zz, cp],
                    send_sem=send_sems1.at[kp, zi],
                    recv_sem=recv_sems1.at[zz, cp],
                    device_id=(z * N_PLANE + cp,),
                    device_id_type=pl.DeviceIdType.MESH,
                )
                recv.wait_recv()
            s = jnp.sum(recv1[zz].astype(jnp.float32), axis=0)
            if zi < N_Z - 1:
                send2[zi] = s.astype(jnp.bfloat16)
                rdma = pltpu.make_async_remote_copy(
                    src_ref=send2.at[zi],
                    dst_ref=recv2.at[z],
                    send_sem=send_sems2.at[zi],
                    recv_sem=recv_sems2.at[z],
                    device_id=(zz * N_PLANE + c,),
                    device_id_type=pl.DeviceIdType.MESH,
                )
                rdma.start()
                sends.append(rdma)
            else:
                recv2[z] = s.astype(jnp.bfloat16)

        for kz in range(1, N_Z):
            zp = (z + kz) % N_Z
            recv = pltpu.make_async_remote_copy(
                src_ref=send2.at[0],
                dst_ref=recv2.at[zp],
                send_sem=send_sems2.at[0],
                recv_sem=recv_sems2.at[zp],
                device_id=(zp * N_PLANE + c,),
                device_id_type=pl.DeviceIdType.MESH,
            )
            recv.wait_recv()

        for rdma in sends:
            rdma.wait_send()

        out_ref[:, :] = jnp.sum(recv2[:, :, :].astype(jnp.float32), axis=0)

    return pl.pallas_call(
        body,
        out_shape=jax.ShapeDtypeStruct((M_CHUNK, n), jnp.float32),
        in_specs=[
            pl.BlockSpec(memory_space=pltpu.VMEM),
            pl.BlockSpec(memory_space=pltpu.VMEM),
        ],
        out_specs=pl.BlockSpec(memory_space=pltpu.VMEM),
        scratch_shapes=[
            pltpu.VMEM((N_PLANE - 1, N_Z, M_CHUNK, n), jnp.bfloat16),
            pltpu.VMEM((N_Z, N_PLANE, M_CHUNK, n), jnp.bfloat16),
            pltpu.VMEM((N_Z - 1, M_CHUNK, n), jnp.bfloat16),
            pltpu.VMEM((N_Z, M_CHUNK, n), jnp.bfloat16),
            pltpu.SemaphoreType.DMA((N_PLANE - 1, N_Z)),
            pltpu.SemaphoreType.DMA((N_Z, N_PLANE)),
            pltpu.SemaphoreType.DMA((N_Z - 1,)),
            pltpu.SemaphoreType.DMA((N_Z,)),
        ],
        compiler_params=pltpu.CompilerParams(collective_id=0),
    )(A, B)
